# baseline (speedup 1.0000x reference)
import jax
import jax.numpy as jnp
from jax.experimental import pallas as pl
from jax.experimental.pallas import tpu as pltpu

_N_CHUNKS = 4


def _dma_copy(src_ref, out_ref, sems):
    rows = out_ref.shape[0]
    chunk = rows // _N_CHUNKS
    copies = [
        pltpu.make_async_copy(
            src_ref.at[pl.ds(i * chunk, chunk), :],
            out_ref.at[pl.ds(i * chunk, chunk), :],
            sems.at[i],
        )
        for i in range(_N_CHUNKS)
    ]
    for c in copies:
        c.start()
    for c in copies:
        c.wait()


def kernel(x, pos_emb):
    seq_len = x.shape[1]
    dim = pos_emb.shape[1]
    return pl.pallas_call(
        _dma_copy,
        in_specs=[pl.BlockSpec(memory_space=pl.ANY)],
        out_specs=pl.BlockSpec(memory_space=pl.ANY),
        scratch_shapes=[pltpu.SemaphoreType.DMA((_N_CHUNKS,))],
        out_shape=jax.ShapeDtypeStruct((seq_len, dim), pos_emb.dtype),
    )(pos_emb)


# speedup vs baseline: 48.4709x; 48.4709x over previous
import jax
import jax.numpy as jnp
from jax.experimental import pallas as pl


def _copy_block(src_ref, out_ref):
    out_ref[...] = src_ref[...]


def kernel(x, pos_emb):
    seq_len = x.shape[1]
    dim = pos_emb.shape[1]
    block = 1024
    grid = (seq_len // block,)
    return pl.pallas_call(
        _copy_block,
        grid=grid,
        in_specs=[pl.BlockSpec((block, dim), lambda i: (i, 0))],
        out_specs=pl.BlockSpec((block, dim), lambda i: (i, 0)),
        out_shape=jax.ShapeDtypeStruct((seq_len, dim), pos_emb.dtype),
    )(pos_emb)
